# Initial kernel scaffold; baseline (speedup 1.0000x reference)
import functools

import jax
import jax.numpy as jnp
from jax import lax
from jax.experimental import pallas as pl
from jax.experimental.pallas import tpu as pltpu
from jax.experimental.pallas import tpu_sc as plsc

N = 10000
IN_DIM = 128
HID_DIM = 64

NC = 2
NS = 16
NW = NC * NS
L = 16

CHUNK = 128
N_ACC = N + 16
ROWS_PER_TILE = N_ACC // NS


def _flat_tile_id():
    return lax.axis_index("c") * NS + lax.axis_index("s")


def _deg_body(nch16, dst_hbm, out_hbm, dst_v, acc):
    wid = _flat_tile_id()
    pltpu.sync_copy(dst_hbm.at[wid], dst_v)

    zero16 = jnp.zeros((L,), jnp.float32)

    @pl.loop(0, N_ACC // L)
    def _(i):
        acc[pl.ds(i * L, L)] = zero16

    ones16 = jnp.ones((L,), jnp.float32)

    @pl.loop(0, nch16)
    def _(k):
        idx = dst_v[k]
        plsc.addupdate_scatter(acc, [idx], ones16)

    pltpu.sync_copy(acc, out_hbm.at[wid])


def _make_deg_call(nch16):
    mesh = plsc.VectorSubcoreMesh(core_axis_name="c", subcore_axis_name="s")
    return pl.kernel(
        functools.partial(_deg_body, nch16),
        out_type=jax.ShapeDtypeStruct((NW, N_ACC), jnp.float32),
        mesh=mesh,
        scratch_types=[
            pltpu.VMEM((nch16, L), jnp.int32),
            pltpu.VMEM((N_ACC,), jnp.float32),
        ],
    )


def _prop_body(nchunk, hs_hbm, src_hbm, dst_hbm, out_hbm,
               src_v, dst_v, buf0, buf1, zrow, accum, gsem):
    c = lax.axis_index("c")
    s = lax.axis_index("s")
    wid = c * NS + s

    pltpu.sync_copy(src_hbm.at[wid], src_v)
    pltpu.sync_copy(dst_hbm.at[wid], dst_v)

    zero16 = jnp.zeros((L,), jnp.float32)

    @pl.loop(0, CHUNK)
    def _(r):
        for q in range(HID_DIM // L):
            zrow[r, pl.ds(q * L, L)] = zero16

    base = s * ROWS_PER_TILE
    nfull = ROWS_PER_TILE // CHUNK
    rem = ROWS_PER_TILE - nfull * CHUNK
    for p in range(nfull):
        pltpu.sync_copy(zrow, accum.at[pl.ds(base + p * CHUNK, CHUNK)])
    if rem:
        pltpu.sync_copy(zrow.at[pl.ds(0, rem)],
                        accum.at[pl.ds(base + nfull * CHUNK, rem)])

    plsc.subcore_barrier()

    pltpu.async_copy(hs_hbm.at[src_v.at[0]], buf0, gsem)

    half = nchunk // 2

    @pl.loop(0, half)
    def _(t):
        j0 = 2 * t
        j1 = j0 + 1
        j2 = jnp.where(t == half - 1, 0, j0 + 2)
        pltpu.make_async_copy(hs_hbm.at[src_v.at[j0]], buf0, gsem).wait()
        pltpu.async_copy(hs_hbm.at[src_v.at[j1]], buf1, gsem)
        pltpu.sync_copy(buf0, accum.at[dst_v.at[j0]], add=True)
        pltpu.make_async_copy(hs_hbm.at[src_v.at[j1]], buf1, gsem).wait()
        pltpu.async_copy(hs_hbm.at[src_v.at[j2]], buf0, gsem)
        pltpu.sync_copy(buf1, accum.at[dst_v.at[j1]], add=True)

    pltpu.make_async_copy(hs_hbm.at[src_v.at[0]], buf0, gsem).wait()

    plsc.subcore_barrier()

    pltpu.sync_copy(accum.at[pl.ds(base, ROWS_PER_TILE)],
                    out_hbm.at[c, pl.ds(base, ROWS_PER_TILE)])


def _make_prop_call(nchunk):
    mesh = plsc.VectorSubcoreMesh(core_axis_name="c", subcore_axis_name="s")
    return pl.kernel(
        functools.partial(_prop_body, nchunk),
        out_type=jax.ShapeDtypeStruct((NC, N_ACC, HID_DIM), jnp.float32),
        mesh=mesh,
        scratch_types=[
            pltpu.VMEM((nchunk, CHUNK), jnp.int32),
            pltpu.VMEM((nchunk, CHUNK), jnp.int32),
            pltpu.VMEM((CHUNK, HID_DIM), jnp.float32),
            pltpu.VMEM((CHUNK, HID_DIM), jnp.float32),
            pltpu.VMEM((CHUNK, HID_DIM), jnp.float32),
            pltpu.VMEM_SHARED((N_ACC, HID_DIM), jnp.float32),
            pltpu.SemaphoreType.DMA,
        ],
    )


ROW_BLK = 1000
GRID = N // ROW_BLK


def _dinv_from_parts(degp):
    deg = jnp.sum(degp, axis=0) + 1.0
    return lax.rsqrt(deg)[:, None]


def _m1_body(x_ref, w1_ref, degp_ref, hs_ref):
    dinv = _dinv_from_parts(degp_ref[...])
    hs_ref[...] = dinv * jnp.dot(x_ref[...], w1_ref[...],
                                 preferred_element_type=jnp.float32)


def _m2_body(s1_ref, hs1_ref, degp_ref, w2_ref, b1_ref, hs2_ref):
    dinv = _dinv_from_parts(degp_ref[...])
    h1 = dinv * (s1_ref[0] + s1_ref[1] + hs1_ref[...]) + b1_ref[...]
    hs2_ref[...] = dinv * jnp.dot(h1, w2_ref[...],
                                  preferred_element_type=jnp.float32)


def _m3_body(s2_ref, hs2_ref, degp_ref, b2_ref, out_ref):
    dinv = _dinv_from_parts(degp_ref[...])
    out_ref[...] = dinv * (s2_ref[0] + s2_ref[1] + hs2_ref[...]) + b2_ref[...]


_m1_call = pl.pallas_call(
    _m1_body,
    grid=(GRID,),
    in_specs=[
        pl.BlockSpec((ROW_BLK, IN_DIM), lambda i: (i, 0)),
        pl.BlockSpec((IN_DIM, HID_DIM), lambda i: (0, 0)),
        pl.BlockSpec((NW, ROW_BLK), lambda i: (0, i)),
    ],
    out_specs=pl.BlockSpec((ROW_BLK, HID_DIM), lambda i: (i, 0)),
    out_shape=jax.ShapeDtypeStruct((N, HID_DIM), jnp.float32),
)

_m2_call = pl.pallas_call(
    _m2_body,
    grid=(GRID,),
    in_specs=[
        pl.BlockSpec((NC, ROW_BLK, HID_DIM), lambda i: (0, i, 0)),
        pl.BlockSpec((ROW_BLK, HID_DIM), lambda i: (i, 0)),
        pl.BlockSpec((NW, ROW_BLK), lambda i: (0, i)),
        pl.BlockSpec((HID_DIM, HID_DIM), lambda i: (0, 0)),
        pl.BlockSpec((1, HID_DIM), lambda i: (0, 0)),
    ],
    out_specs=pl.BlockSpec((ROW_BLK, HID_DIM), lambda i: (i, 0)),
    out_shape=jax.ShapeDtypeStruct((N, HID_DIM), jnp.float32),
)

_m3_call = pl.pallas_call(
    _m3_body,
    grid=(GRID,),
    in_specs=[
        pl.BlockSpec((NC, ROW_BLK, HID_DIM), lambda i: (0, i, 0)),
        pl.BlockSpec((ROW_BLK, HID_DIM), lambda i: (i, 0)),
        pl.BlockSpec((NW, ROW_BLK), lambda i: (0, i)),
        pl.BlockSpec((1, HID_DIM), lambda i: (0, 0)),
    ],
    out_specs=pl.BlockSpec((ROW_BLK, HID_DIM), lambda i: (i, 0)),
    out_shape=jax.ShapeDtypeStruct((N, HID_DIM), jnp.float32),
)


@jax.jit
def kernel(x, edge_index, W1, b1, W2, b2):
    E = edge_index.shape[1]
    e_pad = ((E + NW * CHUNK - 1) // (NW * CHUNK)) * (NW * CHUNK)
    nchunk = e_pad // (NW * CHUNK)
    pad = e_pad - E

    src = edge_index[0]
    dst = edge_index[1]
    src_p = jnp.concatenate([src, jnp.zeros((pad,), src.dtype)])
    dst_p = jnp.concatenate([dst, jnp.full((pad,), N, dst.dtype)])
    src_g = src_p.reshape(NW, nchunk, CHUNK)
    dst_g = dst_p.reshape(NW, nchunk, CHUNK)
    dst_d = dst_p.reshape(NW, e_pad // (NW * L), L)

    degp = _make_deg_call(e_pad // (NW * L))(dst_d)
    prop = _make_prop_call(nchunk)

    hs1 = _m1_call(x, W1, degp)
    s1 = prop(hs1, src_g, dst_g)
    hs2 = _m2_call(s1, hs1, degp, W2, b1.reshape(1, HID_DIM))
    s2 = prop(hs2, src_g, dst_g)
    out = _m3_call(s2, hs2, degp, b2.reshape(1, HID_DIM))
    return out


# speedup vs baseline: 15.0385x; 15.0385x over previous
import functools

import jax
import jax.numpy as jnp
from jax import lax
from jax.experimental import pallas as pl
from jax.experimental.pallas import tpu as pltpu
from jax.experimental.pallas import tpu_sc as plsc

N = 10000
IN_DIM = 128
HID_DIM = 64

NC = 2
NS = 16
NW = NC * NS
L = 16

CHUNK = 128
N_ACC = 10112
ROWS_PER_TILE = N_ACC // NS


def _flat_tile_id():
    return lax.axis_index("c") * NS + lax.axis_index("s")


def _deg_body(nch16, dst_hbm, out_hbm, dst_v, acc):
    wid = _flat_tile_id()
    pltpu.sync_copy(dst_hbm.at[wid], dst_v)

    zero16 = jnp.zeros((L,), jnp.float32)

    @pl.loop(0, N_ACC // L)
    def _(i):
        acc[pl.ds(i * L, L)] = zero16

    ones16 = jnp.ones((L,), jnp.float32)

    @pl.loop(0, nch16)
    def _(k):
        idx = dst_v[k]
        plsc.addupdate_scatter(acc, [idx], ones16)

    pltpu.sync_copy(acc, out_hbm.at[wid])


def _make_deg_call(nch16):
    mesh = plsc.VectorSubcoreMesh(core_axis_name="c", subcore_axis_name="s")
    return pl.kernel(
        functools.partial(_deg_body, nch16),
        out_type=jax.ShapeDtypeStruct((NW, N_ACC), jnp.float32),
        mesh=mesh,
        scratch_types=[
            pltpu.VMEM((nch16, L), jnp.int32),
            pltpu.VMEM((N_ACC,), jnp.float32),
        ],
        compiler_params=pltpu.CompilerParams(needs_layout_passes=False),
    )


def _prop_body(nchunk, hs_hbm, src_hbm, dst_hbm, out_hbm,
               src_v, dst_v, buf0, buf1, zrow, accum, gsem0, gsem1):
    c = lax.axis_index("c")
    s = lax.axis_index("s")
    wid = c * NS + s

    pltpu.sync_copy(src_hbm.at[wid], src_v)
    pltpu.sync_copy(dst_hbm.at[wid], dst_v)

    zero16 = jnp.zeros((L,), jnp.float32)

    @pl.loop(0, CHUNK)
    def _(r):
        for q in range(HID_DIM // L):
            zrow[r, pl.ds(q * L, L)] = zero16

    base = s * ROWS_PER_TILE
    nfull = ROWS_PER_TILE // CHUNK
    rem = ROWS_PER_TILE - nfull * CHUNK
    for p in range(nfull):
        pltpu.sync_copy(zrow, accum.at[pl.ds(base + p * CHUNK, CHUNK)])
    if rem:
        pltpu.sync_copy(zrow.at[pl.ds(0, rem)],
                        accum.at[pl.ds(base + nfull * CHUNK, rem)])

    plsc.subcore_barrier()

    half = nchunk // 2

    @pl.loop(0, half)
    def _(t):
        j0 = 2 * t
        j1 = j0 + 1
        d0 = pltpu.async_copy(hs_hbm.at[src_v.at[j0]], buf0, gsem0)
        d1 = pltpu.async_copy(hs_hbm.at[src_v.at[j1]], buf1, gsem1)
        d0.wait()
        pltpu.sync_copy(buf0, accum.at[dst_v.at[j0]], add=True)
        d1.wait()
        pltpu.sync_copy(buf1, accum.at[dst_v.at[j1]], add=True)

    plsc.subcore_barrier()

    pltpu.sync_copy(accum.at[pl.ds(base, ROWS_PER_TILE)],
                    out_hbm.at[c, pl.ds(base, ROWS_PER_TILE)])


def _make_prop_call(nchunk):
    mesh = plsc.VectorSubcoreMesh(core_axis_name="c", subcore_axis_name="s")
    return pl.kernel(
        functools.partial(_prop_body, nchunk),
        out_type=jax.ShapeDtypeStruct((NC, N_ACC, HID_DIM), jnp.float32),
        mesh=mesh,
        scratch_types=[
            pltpu.VMEM((nchunk, CHUNK), jnp.int32),
            pltpu.VMEM((nchunk, CHUNK), jnp.int32),
            pltpu.VMEM((CHUNK, HID_DIM), jnp.float32),
            pltpu.VMEM((CHUNK, HID_DIM), jnp.float32),
            pltpu.VMEM((CHUNK, HID_DIM), jnp.float32),
            pltpu.VMEM_SHARED((N_ACC, HID_DIM), jnp.float32),
            pltpu.SemaphoreType.DMA,
            pltpu.SemaphoreType.DMA,
        ],
        compiler_params=pltpu.CompilerParams(use_tc_tiling_on_sc=False),
    )


ROW_BLK = 1000
GRID = N // ROW_BLK


def _dinv_body(degp_ref, dinv_ref):
    deg = jnp.sum(degp_ref[...], axis=0) + 1.0
    dinv_ref[...] = lax.rsqrt(deg)[:, None]


_dinv_call = pl.pallas_call(
    _dinv_body,
    out_shape=jax.ShapeDtypeStruct((N_ACC, 1), jnp.float32),
)


def _m1_body(x_ref, w1_ref, dinv_ref, hs_ref):
    hs_ref[...] = dinv_ref[...] * jnp.dot(x_ref[...], w1_ref[...],
                                          preferred_element_type=jnp.float32)


def _m2_body(s1_ref, hs1_ref, dinv_ref, w2_ref, b1_ref, hs2_ref):
    dinv = dinv_ref[...]
    h1 = dinv * (s1_ref[0] + s1_ref[1] + hs1_ref[...]) + b1_ref[...]
    hs2_ref[...] = dinv * jnp.dot(h1, w2_ref[...],
                                  preferred_element_type=jnp.float32)


def _m3_body(s2_ref, hs2_ref, dinv_ref, b2_ref, out_ref):
    dinv = dinv_ref[...]
    out_ref[...] = dinv * (s2_ref[0] + s2_ref[1] + hs2_ref[...]) + b2_ref[...]


_m1_call = pl.pallas_call(
    _m1_body,
    grid=(GRID,),
    in_specs=[
        pl.BlockSpec((ROW_BLK, IN_DIM), lambda i: (i, 0)),
        pl.BlockSpec((IN_DIM, HID_DIM), lambda i: (0, 0)),
        pl.BlockSpec((ROW_BLK, 1), lambda i: (i, 0)),
    ],
    out_specs=pl.BlockSpec((ROW_BLK, HID_DIM), lambda i: (i, 0)),
    out_shape=jax.ShapeDtypeStruct((N, HID_DIM), jnp.float32),
)

_m2_call = pl.pallas_call(
    _m2_body,
    grid=(GRID,),
    in_specs=[
        pl.BlockSpec((NC, ROW_BLK, HID_DIM), lambda i: (0, i, 0)),
        pl.BlockSpec((ROW_BLK, HID_DIM), lambda i: (i, 0)),
        pl.BlockSpec((ROW_BLK, 1), lambda i: (i, 0)),
        pl.BlockSpec((HID_DIM, HID_DIM), lambda i: (0, 0)),
        pl.BlockSpec((1, HID_DIM), lambda i: (0, 0)),
    ],
    out_specs=pl.BlockSpec((ROW_BLK, HID_DIM), lambda i: (i, 0)),
    out_shape=jax.ShapeDtypeStruct((N, HID_DIM), jnp.float32),
)

_m3_call = pl.pallas_call(
    _m3_body,
    grid=(GRID,),
    in_specs=[
        pl.BlockSpec((NC, ROW_BLK, HID_DIM), lambda i: (0, i, 0)),
        pl.BlockSpec((ROW_BLK, HID_DIM), lambda i: (i, 0)),
        pl.BlockSpec((ROW_BLK, 1), lambda i: (i, 0)),
        pl.BlockSpec((1, HID_DIM), lambda i: (0, 0)),
    ],
    out_specs=pl.BlockSpec((ROW_BLK, HID_DIM), lambda i: (i, 0)),
    out_shape=jax.ShapeDtypeStruct((N, HID_DIM), jnp.float32),
)


@jax.jit
def kernel(x, edge_index, W1, b1, W2, b2):
    E = edge_index.shape[1]
    grain = NW * CHUNK * 2
    e_pad = ((E + grain - 1) // grain) * grain
    nchunk = e_pad // (NW * CHUNK)
    pad = e_pad - E

    src = edge_index[0]
    dst = edge_index[1]
    src_p = jnp.concatenate([src, jnp.zeros((pad,), src.dtype)])
    dst_p = jnp.concatenate([dst, jnp.full((pad,), N, dst.dtype)])
    src_g = src_p.reshape(NW, nchunk, CHUNK)
    dst_g = dst_p.reshape(NW, nchunk, CHUNK)
    dst_d = dst_p.reshape(NW, e_pad // (NW * L), L)

    degp = _make_deg_call(e_pad // (NW * L))(dst_d)
    dinv = _dinv_call(degp)
    prop = _make_prop_call(nchunk)

    hs1 = _m1_call(x, W1, dinv)
    s1 = prop(hs1, src_g, dst_g)
    hs2 = _m2_call(s1, hs1, dinv, W2, b1.reshape(1, HID_DIM))
    s2 = prop(hs2, src_g, dst_g)
    out = _m3_call(s2, hs2, dinv, b2.reshape(1, HID_DIM))
    return out


# speedup vs baseline: 15.6752x; 1.0423x over previous
import functools

import jax
import jax.numpy as jnp
from jax import lax
from jax.experimental import pallas as pl
from jax.experimental.pallas import tpu as pltpu
from jax.experimental.pallas import tpu_sc as plsc

N = 10000
IN_DIM = 128
HID_DIM = 64

NC = 2
NS = 16
NW = NC * NS
L = 16

CHUNK = 128
N_ACC = 10112
ROWS_PER_TILE = N_ACC // NS


def _flat_tile_id():
    return lax.axis_index("c") * NS + lax.axis_index("s")


def _deg_body(nch16, dst_hbm, out_hbm, dst_v, acc):
    wid = _flat_tile_id()
    pltpu.sync_copy(dst_hbm.at[wid], dst_v)

    zero16 = jnp.zeros((L,), jnp.float32)

    @pl.loop(0, N_ACC // L)
    def _(i):
        acc[pl.ds(i * L, L)] = zero16

    ones16 = jnp.ones((L,), jnp.float32)

    @pl.loop(0, nch16)
    def _(k):
        idx = dst_v[k]
        plsc.addupdate_scatter(acc, [idx], ones16)

    pltpu.sync_copy(acc, out_hbm.at[wid])


def _make_deg_call(nch16):
    mesh = plsc.VectorSubcoreMesh(core_axis_name="c", subcore_axis_name="s")
    return pl.kernel(
        functools.partial(_deg_body, nch16),
        out_type=jax.ShapeDtypeStruct((NW, N_ACC), jnp.float32),
        mesh=mesh,
        scratch_types=[
            pltpu.VMEM((nch16, L), jnp.int32),
            pltpu.VMEM((N_ACC,), jnp.float32),
        ],
        compiler_params=pltpu.CompilerParams(needs_layout_passes=False),
    )


def _prop_body(nchunk, hs_hbm, src_hbm, dst_hbm, out_hbm,
               src_v, dst_v, buf0, buf1, zrow, accum, gsem0, gsem1):
    c = lax.axis_index("c")
    s = lax.axis_index("s")
    wid = c * NS + s

    pltpu.sync_copy(src_hbm.at[wid], src_v)
    pltpu.sync_copy(dst_hbm.at[wid], dst_v)

    zero16 = jnp.zeros((L,), jnp.float32)

    @pl.loop(0, CHUNK)
    def _(r):
        for q in range(HID_DIM // L):
            zrow[r, pl.ds(q * L, L)] = zero16

    base = s * ROWS_PER_TILE
    nfull = ROWS_PER_TILE // CHUNK
    rem = ROWS_PER_TILE - nfull * CHUNK
    for p in range(nfull):
        pltpu.sync_copy(zrow, accum.at[pl.ds(base + p * CHUNK, CHUNK)])
    if rem:
        pltpu.sync_copy(zrow.at[pl.ds(0, rem)],
                        accum.at[pl.ds(base + nfull * CHUNK, rem)])

    plsc.subcore_barrier()

    half = nchunk // 2

    @pl.loop(0, half)
    def _(t):
        j0 = 2 * t
        j1 = j0 + 1
        d0 = pltpu.async_copy(hs_hbm.at[src_v.at[j0]], buf0, gsem0)
        d1 = pltpu.async_copy(hs_hbm.at[src_v.at[j1]], buf1, gsem1)
        d0.wait()
        pltpu.sync_copy(buf0, accum.at[dst_v.at[j0]], add=True)
        d1.wait()
        pltpu.sync_copy(buf1, accum.at[dst_v.at[j1]], add=True)

    plsc.subcore_barrier()

    pltpu.sync_copy(accum.at[pl.ds(base, ROWS_PER_TILE)],
                    out_hbm.at[c, pl.ds(base, ROWS_PER_TILE)])


def _make_prop_call(nchunk):
    mesh = plsc.VectorSubcoreMesh(core_axis_name="c", subcore_axis_name="s")
    return pl.kernel(
        functools.partial(_prop_body, nchunk),
        out_type=jax.ShapeDtypeStruct((NC, N_ACC, HID_DIM), jnp.float32),
        mesh=mesh,
        scratch_types=[
            pltpu.VMEM((nchunk, CHUNK), jnp.int32),
            pltpu.VMEM((nchunk, CHUNK), jnp.int32),
            pltpu.VMEM((CHUNK, HID_DIM), jnp.float32),
            pltpu.VMEM((CHUNK, HID_DIM), jnp.float32),
            pltpu.VMEM((CHUNK, HID_DIM), jnp.float32),
            pltpu.VMEM_SHARED((N_ACC, HID_DIM), jnp.float32),
            pltpu.SemaphoreType.DMA,
            pltpu.SemaphoreType.DMA,
        ],
        compiler_params=pltpu.CompilerParams(use_tc_tiling_on_sc=False),
    )


ROW_BLK = 1000
GRID = N // ROW_BLK


def _dinv_body(degp_ref, dinv_ref):
    deg = jnp.sum(degp_ref[...], axis=0) + 1.0
    dinv_ref[...] = lax.rsqrt(deg)[:, None]


_dinv_call = pl.pallas_call(
    _dinv_body,
    out_shape=jax.ShapeDtypeStruct((N_ACC, 1), jnp.float32),
)


def _m1_body(x_ref, w1_ref, dinv_ref, hs_ref):
    hs_ref[...] = dinv_ref[...] * jnp.dot(x_ref[...], w1_ref[...],
                                          preferred_element_type=jnp.float32)


def _m2_body(s1_ref, hs1_ref, dinv_ref, w2_ref, b1_ref, hs2_ref):
    dinv = dinv_ref[...]
    h1 = dinv * (s1_ref[0] + s1_ref[1] + hs1_ref[...]) + b1_ref[...]
    hs2_ref[...] = dinv * jnp.dot(h1, w2_ref[...],
                                  preferred_element_type=jnp.float32)


def _m3_body(s2_ref, hs2_ref, dinv_ref, b2_ref, out_ref):
    dinv = dinv_ref[...]
    out_ref[...] = dinv * (s2_ref[0] + s2_ref[1] + hs2_ref[...]) + b2_ref[...]


_m1_call = pl.pallas_call(
    _m1_body,
    grid=(GRID,),
    in_specs=[
        pl.BlockSpec((ROW_BLK, IN_DIM), lambda i: (i, 0)),
        pl.BlockSpec((IN_DIM, HID_DIM), lambda i: (0, 0)),
        pl.BlockSpec((ROW_BLK, 1), lambda i: (i, 0)),
    ],
    out_specs=pl.BlockSpec((ROW_BLK, HID_DIM), lambda i: (i, 0)),
    out_shape=jax.ShapeDtypeStruct((N, HID_DIM), jnp.float32),
)

_m2_call = pl.pallas_call(
    _m2_body,
    grid=(GRID,),
    in_specs=[
        pl.BlockSpec((NC, ROW_BLK, HID_DIM), lambda i: (0, i, 0)),
        pl.BlockSpec((ROW_BLK, HID_DIM), lambda i: (i, 0)),
        pl.BlockSpec((ROW_BLK, 1), lambda i: (i, 0)),
        pl.BlockSpec((HID_DIM, HID_DIM), lambda i: (0, 0)),
        pl.BlockSpec((1, HID_DIM), lambda i: (0, 0)),
    ],
    out_specs=pl.BlockSpec((ROW_BLK, HID_DIM), lambda i: (i, 0)),
    out_shape=jax.ShapeDtypeStruct((N, HID_DIM), jnp.float32),
)

_m3_call = pl.pallas_call(
    _m3_body,
    grid=(GRID,),
    in_specs=[
        pl.BlockSpec((NC, ROW_BLK, HID_DIM), lambda i: (0, i, 0)),
        pl.BlockSpec((ROW_BLK, HID_DIM), lambda i: (i, 0)),
        pl.BlockSpec((ROW_BLK, 1), lambda i: (i, 0)),
        pl.BlockSpec((1, HID_DIM), lambda i: (0, 0)),
    ],
    out_specs=pl.BlockSpec((ROW_BLK, HID_DIM), lambda i: (i, 0)),
    out_shape=jax.ShapeDtypeStruct((N, HID_DIM), jnp.float32),
)


@jax.jit
def kernel(x, edge_index, W1, b1, W2, b2):
    E = edge_index.shape[1]
    grain = NW * CHUNK * 2
    e_pad = ((E + grain - 1) // grain) * grain
    nchunk = e_pad // (NW * CHUNK)
    pad = e_pad - E

    src = edge_index[0]
    dst = edge_index[1]
    dump = N + jax.lax.rem(jnp.arange(pad, dtype=dst.dtype),
                           jnp.asarray(N_ACC - N, dst.dtype))
    src_p = jnp.concatenate([src, jnp.zeros((pad,), src.dtype)])
    dst_p = jnp.concatenate([dst, dump])
    src_g = src_p.reshape(NW, nchunk, CHUNK)
    dst_g = dst_p.reshape(NW, nchunk, CHUNK)
    dst_d = dst_p.reshape(NW, e_pad // (NW * L), L)

    degp = _make_deg_call(e_pad // (NW * L))(dst_d)
    dinv = _dinv_call(degp)
    prop = _make_prop_call(nchunk)

    hs1 = _m1_call(x, W1, dinv)
    s1 = prop(hs1, src_g, dst_g)
    hs2 = _m2_call(s1, hs1, dinv, W2, b1.reshape(1, HID_DIM))
    s2 = prop(hs2, src_g, dst_g)
    out = _m3_call(s2, hs2, dinv, b2.reshape(1, HID_DIM))
    return out


# speedup vs baseline: 16.8300x; 1.0737x over previous
import functools

import jax
import jax.numpy as jnp
from jax import lax
from jax.experimental import pallas as pl
from jax.experimental.pallas import tpu as pltpu
from jax.experimental.pallas import tpu_sc as plsc

N = 10000
IN_DIM = 128
HID_DIM = 64

NC = 2
NS = 16
NW = NC * NS
L = 16

CHUNK = 128
N_ACC = 10112
ROWS_PER_TILE = N_ACC // NS


def _flat_tile_id():
    return lax.axis_index("c") * NS + lax.axis_index("s")


def _deg_body(nch16, dst_hbm, out_hbm, dst_v, acc):
    wid = _flat_tile_id()
    pltpu.sync_copy(dst_hbm.at[wid], dst_v)

    zero16 = jnp.zeros((L,), jnp.float32)

    @pl.loop(0, N_ACC // L)
    def _(i):
        acc[pl.ds(i * L, L)] = zero16

    ones16 = jnp.ones((L,), jnp.float32)

    @pl.loop(0, nch16)
    def _(k):
        idx = dst_v[k]
        plsc.addupdate_scatter(acc, [idx], ones16)

    pltpu.sync_copy(acc, out_hbm.at[wid])


def _make_deg_call(nch16):
    mesh = plsc.VectorSubcoreMesh(core_axis_name="c", subcore_axis_name="s")
    return pl.kernel(
        functools.partial(_deg_body, nch16),
        out_type=jax.ShapeDtypeStruct((NW, N_ACC), jnp.float32),
        mesh=mesh,
        scratch_types=[
            pltpu.VMEM((nch16, L), jnp.int32),
            pltpu.VMEM((N_ACC,), jnp.float32),
        ],
        compiler_params=pltpu.CompilerParams(needs_layout_passes=False),
    )


DEPTH = 8


def _prop_body(nchunk, hs_hbm, src_hbm, dst_hbm, out_hbm,
               src_v, dst_v, bufs, accum, gsems, ssems):
    c = lax.axis_index("c")
    s = lax.axis_index("s")
    wid = c * NS + s

    pltpu.sync_copy(src_hbm.at[wid], src_v)
    pltpu.sync_copy(dst_hbm.at[wid], dst_v)

    zero16 = jnp.zeros((L,), jnp.float32)
    zrow = bufs.at[0]

    @pl.loop(0, CHUNK)
    def _(r):
        for q in range(HID_DIM // L):
            zrow[r, pl.ds(q * L, L)] = zero16

    base = s * ROWS_PER_TILE
    nfull = ROWS_PER_TILE // CHUNK
    rem = ROWS_PER_TILE - nfull * CHUNK
    for p in range(nfull):
        pltpu.sync_copy(zrow, accum.at[pl.ds(base + p * CHUNK, CHUNK)])
    if rem:
        pltpu.sync_copy(zrow.at[pl.ds(0, rem)],
                        accum.at[pl.ds(base + nfull * CHUNK, rem)])

    plsc.subcore_barrier()

    @pl.loop(0, nchunk // DEPTH)
    def _(t):
        j0 = DEPTH * t
        gd = [pltpu.async_copy(hs_hbm.at[src_v.at[j0 + k]], bufs.at[k],
                               gsems.at[k])
              for k in range(DEPTH)]
        sd = []
        for k in range(DEPTH):
            gd[k].wait()
            sd.append(pltpu.async_copy(bufs.at[k],
                                       accum.at[dst_v.at[j0 + k]],
                                       ssems.at[k], add=True))
        for k in range(DEPTH):
            sd[k].wait()

    plsc.subcore_barrier()

    pltpu.sync_copy(accum.at[pl.ds(base, ROWS_PER_TILE)],
                    out_hbm.at[c, pl.ds(base, ROWS_PER_TILE)])


def _make_prop_call(nchunk):
    mesh = plsc.VectorSubcoreMesh(core_axis_name="c", subcore_axis_name="s")
    return pl.kernel(
        functools.partial(_prop_body, nchunk),
        out_type=jax.ShapeDtypeStruct((NC, N_ACC, HID_DIM), jnp.float32),
        mesh=mesh,
        scratch_types=[
            pltpu.VMEM((nchunk, CHUNK), jnp.int32),
            pltpu.VMEM((nchunk, CHUNK), jnp.int32),
            pltpu.VMEM((DEPTH, CHUNK, HID_DIM), jnp.float32),
            pltpu.VMEM_SHARED((N_ACC, HID_DIM), jnp.float32),
            pltpu.SemaphoreType.DMA((DEPTH,)),
            pltpu.SemaphoreType.DMA((DEPTH,)),
        ],
        compiler_params=pltpu.CompilerParams(use_tc_tiling_on_sc=False),
    )


ROW_BLK = 1000
GRID = N // ROW_BLK


def _dinv_body(degp_ref, dinv_ref):
    deg = jnp.sum(degp_ref[...], axis=0) + 1.0
    dinv_ref[...] = lax.rsqrt(deg)[:, None]


_dinv_call = pl.pallas_call(
    _dinv_body,
    out_shape=jax.ShapeDtypeStruct((N_ACC, 1), jnp.float32),
)


def _m1_body(x_ref, w1_ref, dinv_ref, hs_ref):
    hs_ref[...] = dinv_ref[...] * jnp.dot(x_ref[...], w1_ref[...],
                                          preferred_element_type=jnp.float32)


def _m2_body(s1_ref, hs1_ref, dinv_ref, w2_ref, b1_ref, hs2_ref):
    dinv = dinv_ref[...]
    h1 = dinv * (s1_ref[0] + s1_ref[1] + hs1_ref[...]) + b1_ref[...]
    hs2_ref[...] = dinv * jnp.dot(h1, w2_ref[...],
                                  preferred_element_type=jnp.float32)


def _m3_body(s2_ref, hs2_ref, dinv_ref, b2_ref, out_ref):
    dinv = dinv_ref[...]
    out_ref[...] = dinv * (s2_ref[0] + s2_ref[1] + hs2_ref[...]) + b2_ref[...]


_m1_call = pl.pallas_call(
    _m1_body,
    grid=(GRID,),
    in_specs=[
        pl.BlockSpec((ROW_BLK, IN_DIM), lambda i: (i, 0)),
        pl.BlockSpec((IN_DIM, HID_DIM), lambda i: (0, 0)),
        pl.BlockSpec((ROW_BLK, 1), lambda i: (i, 0)),
    ],
    out_specs=pl.BlockSpec((ROW_BLK, HID_DIM), lambda i: (i, 0)),
    out_shape=jax.ShapeDtypeStruct((N, HID_DIM), jnp.float32),
)

_m2_call = pl.pallas_call(
    _m2_body,
    grid=(GRID,),
    in_specs=[
        pl.BlockSpec((NC, ROW_BLK, HID_DIM), lambda i: (0, i, 0)),
        pl.BlockSpec((ROW_BLK, HID_DIM), lambda i: (i, 0)),
        pl.BlockSpec((ROW_BLK, 1), lambda i: (i, 0)),
        pl.BlockSpec((HID_DIM, HID_DIM), lambda i: (0, 0)),
        pl.BlockSpec((1, HID_DIM), lambda i: (0, 0)),
    ],
    out_specs=pl.BlockSpec((ROW_BLK, HID_DIM), lambda i: (i, 0)),
    out_shape=jax.ShapeDtypeStruct((N, HID_DIM), jnp.float32),
)

_m3_call = pl.pallas_call(
    _m3_body,
    grid=(GRID,),
    in_specs=[
        pl.BlockSpec((NC, ROW_BLK, HID_DIM), lambda i: (0, i, 0)),
        pl.BlockSpec((ROW_BLK, HID_DIM), lambda i: (i, 0)),
        pl.BlockSpec((ROW_BLK, 1), lambda i: (i, 0)),
        pl.BlockSpec((1, HID_DIM), lambda i: (0, 0)),
    ],
    out_specs=pl.BlockSpec((ROW_BLK, HID_DIM), lambda i: (i, 0)),
    out_shape=jax.ShapeDtypeStruct((N, HID_DIM), jnp.float32),
)


@jax.jit
def kernel(x, edge_index, W1, b1, W2, b2):
    E = edge_index.shape[1]
    grain = NW * CHUNK * DEPTH
    e_pad = ((E + grain - 1) // grain) * grain
    nchunk = e_pad // (NW * CHUNK)
    pad = e_pad - E

    src = edge_index[0]
    dst = edge_index[1]
    dump = N + jax.lax.rem(jnp.arange(pad, dtype=dst.dtype),
                           jnp.asarray(N_ACC - N, dst.dtype))
    src_p = jnp.concatenate([src, jnp.zeros((pad,), src.dtype)])
    dst_p = jnp.concatenate([dst, dump])
    src_g = src_p.reshape(NW, nchunk, CHUNK)
    dst_g = dst_p.reshape(NW, nchunk, CHUNK)
    dst_d = dst_p.reshape(NW, e_pad // (NW * L), L)

    degp = _make_deg_call(e_pad // (NW * L))(dst_d)
    dinv = _dinv_call(degp)
    prop = _make_prop_call(nchunk)

    hs1 = _m1_call(x, W1, dinv)
    s1 = prop(hs1, src_g, dst_g)
    hs2 = _m2_call(s1, hs1, dinv, W2, b1.reshape(1, HID_DIM))
    s2 = prop(hs2, src_g, dst_g)
    out = _m3_call(s2, hs2, dinv, b2.reshape(1, HID_DIM))
    return out


# speedup vs baseline: 39.1119x; 2.3239x over previous
import functools

import jax
import jax.numpy as jnp
from jax import lax
from jax.experimental import pallas as pl
from jax.experimental.pallas import tpu as pltpu
from jax.experimental.pallas import tpu_sc as plsc

N = 10000
IN_DIM = 128
HID_DIM = 64

NC = 2
NS = 16
NW = NC * NS
L = 16

CHUNK = 128
N_ACC = 10112
ROWS_PER_TILE = N_ACC // NS


def _flat_tile_id():
    return lax.axis_index("c") * NS + lax.axis_index("s")


def _deg_body(nch16, dst_hbm, out_hbm, dst_v, acc):
    wid = _flat_tile_id()
    pltpu.sync_copy(dst_hbm.at[wid], dst_v)

    zero16 = jnp.zeros((L,), jnp.float32)

    @pl.loop(0, N_ACC // L)
    def _(i):
        acc[pl.ds(i * L, L)] = zero16

    ones16 = jnp.ones((L,), jnp.float32)

    @pl.loop(0, nch16)
    def _(k):
        idx = dst_v[k]
        plsc.addupdate_scatter(acc, [idx], ones16)

    pltpu.sync_copy(acc, out_hbm.at[wid])


def _make_deg_call(nch16):
    mesh = plsc.VectorSubcoreMesh(core_axis_name="c", subcore_axis_name="s")
    return pl.kernel(
        functools.partial(_deg_body, nch16),
        out_type=jax.ShapeDtypeStruct((NW, N_ACC), jnp.float32),
        mesh=mesh,
        scratch_types=[
            pltpu.VMEM((nch16, L), jnp.int32),
            pltpu.VMEM((N_ACC,), jnp.float32),
        ],
        compiler_params=pltpu.CompilerParams(needs_layout_passes=False),
    )


DEPTH = 8


def _prop_body(nchunk, hs_hbm, src_hbm, dst_hbm, out_hbm,
               src_v, dst_v, bufs, accum, gsems, ssems):
    c = lax.axis_index("c")
    s = lax.axis_index("s")
    wid = c * NS + s

    pltpu.sync_copy(src_hbm.at[wid], src_v)
    pltpu.sync_copy(dst_hbm.at[wid], dst_v)

    zero16 = jnp.zeros((L,), jnp.float32)
    zrow = bufs.at[0]

    @pl.loop(0, CHUNK)
    def _(r):
        for q in range(HID_DIM // L):
            zrow[r, pl.ds(q * L, L)] = zero16

    base = s * ROWS_PER_TILE
    nfull = ROWS_PER_TILE // CHUNK
    rem = ROWS_PER_TILE - nfull * CHUNK
    for p in range(nfull):
        pltpu.sync_copy(zrow, accum.at[pl.ds(base + p * CHUNK, CHUNK)])
    if rem:
        pltpu.sync_copy(zrow.at[pl.ds(0, rem)],
                        accum.at[pl.ds(base + nfull * CHUNK, rem)])

    plsc.subcore_barrier()

    @pl.loop(0, nchunk // DEPTH)
    def _(t):
        j0 = DEPTH * t
        gd = [pltpu.async_copy(hs_hbm.at[src_v.at[j0 + k]], bufs.at[k],
                               gsems.at[k])
              for k in range(DEPTH)]
        sd = []
        for k in range(DEPTH):
            gd[k].wait()
            sd.append(pltpu.async_copy(bufs.at[k],
                                       accum.at[dst_v.at[j0 + k]],
                                       ssems.at[k], add=True))
        for k in range(DEPTH):
            sd[k].wait()

    plsc.subcore_barrier()

    pltpu.sync_copy(accum.at[pl.ds(base, ROWS_PER_TILE)],
                    out_hbm.at[c, pl.ds(base, ROWS_PER_TILE)])


def _make_prop_call(nchunk):
    mesh = plsc.VectorSubcoreMesh(core_axis_name="c", subcore_axis_name="s")
    return pl.kernel(
        functools.partial(_prop_body, nchunk),
        out_type=jax.ShapeDtypeStruct((NC, N_ACC, HID_DIM), jnp.float32),
        mesh=mesh,
        scratch_types=[
            pltpu.VMEM((nchunk, CHUNK), jnp.int32),
            pltpu.VMEM((nchunk, CHUNK), jnp.int32),
            pltpu.VMEM((DEPTH, CHUNK, HID_DIM), jnp.float32),
            pltpu.VMEM_SHARED((N_ACC, HID_DIM), jnp.float32),
            pltpu.SemaphoreType.DMA((DEPTH,)),
            pltpu.SemaphoreType.DMA((DEPTH,)),
        ],
        compiler_params=pltpu.CompilerParams(use_tc_tiling_on_sc=False),
    )


ROW_BLK = 1000
GRID = N // ROW_BLK


def _dinv_body(degp_ref, dinv_ref):
    deg = jnp.sum(degp_ref[...], axis=0) + 1.0
    dinv_ref[...] = lax.rsqrt(deg)[:, None]


_dinv_call = pl.pallas_call(
    _dinv_body,
    out_shape=jax.ShapeDtypeStruct((N_ACC, 1), jnp.float32),
)


def _m1_body(x_ref, w1_ref, dinv_ref, hs_ref):
    hs_ref[...] = dinv_ref[...] * jnp.dot(x_ref[...], w1_ref[...],
                                          preferred_element_type=jnp.float32)


def _m2_body(s1_ref, hs1_ref, dinv_ref, w2_ref, b1_ref, hs2_ref):
    dinv = dinv_ref[...]
    h1 = dinv * (s1_ref[0] + s1_ref[1] + hs1_ref[...]) + b1_ref[...]
    hs2_ref[...] = dinv * jnp.dot(h1, w2_ref[...],
                                  preferred_element_type=jnp.float32)


def _m3_body(s2_ref, hs2_ref, dinv_ref, b2_ref, out_ref):
    dinv = dinv_ref[...]
    out_ref[...] = dinv * (s2_ref[0] + s2_ref[1] + hs2_ref[...]) + b2_ref[...]


_m1_call = pl.pallas_call(
    _m1_body,
    grid=(GRID,),
    in_specs=[
        pl.BlockSpec((ROW_BLK, IN_DIM), lambda i: (i, 0)),
        pl.BlockSpec((IN_DIM, HID_DIM), lambda i: (0, 0)),
        pl.BlockSpec((ROW_BLK, 1), lambda i: (i, 0)),
    ],
    out_specs=pl.BlockSpec((ROW_BLK, HID_DIM), lambda i: (i, 0)),
    out_shape=jax.ShapeDtypeStruct((N, HID_DIM), jnp.float32),
)

_m2_call = pl.pallas_call(
    _m2_body,
    grid=(GRID,),
    in_specs=[
        pl.BlockSpec((NC, ROW_BLK, HID_DIM), lambda i: (0, i, 0)),
        pl.BlockSpec((ROW_BLK, HID_DIM), lambda i: (i, 0)),
        pl.BlockSpec((ROW_BLK, 1), lambda i: (i, 0)),
        pl.BlockSpec((HID_DIM, HID_DIM), lambda i: (0, 0)),
        pl.BlockSpec((1, HID_DIM), lambda i: (0, 0)),
    ],
    out_specs=pl.BlockSpec((ROW_BLK, HID_DIM), lambda i: (i, 0)),
    out_shape=jax.ShapeDtypeStruct((N, HID_DIM), jnp.float32),
)

_m3_call = pl.pallas_call(
    _m3_body,
    grid=(GRID,),
    in_specs=[
        pl.BlockSpec((NC, ROW_BLK, HID_DIM), lambda i: (0, i, 0)),
        pl.BlockSpec((ROW_BLK, HID_DIM), lambda i: (i, 0)),
        pl.BlockSpec((ROW_BLK, 1), lambda i: (i, 0)),
        pl.BlockSpec((1, HID_DIM), lambda i: (0, 0)),
    ],
    out_specs=pl.BlockSpec((ROW_BLK, HID_DIM), lambda i: (i, 0)),
    out_shape=jax.ShapeDtypeStruct((N, HID_DIM), jnp.float32),
)


@jax.jit
def kernel(x, edge_index, W1, b1, W2, b2):
    E = edge_index.shape[1]
    grain = NW * CHUNK * DEPTH
    e_pad = ((E + grain - 1) // grain) * grain
    nchunk = e_pad // (NW * CHUNK)
    pad = e_pad - E

    src = edge_index[0]
    dst = edge_index[1]
    ar = jnp.arange(pad, dtype=dst.dtype)
    dump = N + jax.lax.rem(ar, jnp.asarray(N_ACC - N, dst.dtype))
    fake_src = jax.lax.rem(ar * 257, jnp.asarray(N, src.dtype))
    src_p = jnp.concatenate([src, fake_src])
    dst_p = jnp.concatenate([dst, dump])
    src_g = src_p.reshape(NW, nchunk, CHUNK)
    dst_g = dst_p.reshape(NW, nchunk, CHUNK)
    dst_d = dst_p.reshape(NW, e_pad // (NW * L), L)

    degp = _make_deg_call(e_pad // (NW * L))(dst_d)
    dinv = _dinv_call(degp)
    prop = _make_prop_call(nchunk)

    hs1 = _m1_call(x, W1, dinv)
    s1 = prop(hs1, src_g, dst_g)
    hs2 = _m2_call(s1, hs1, dinv, W2, b1.reshape(1, HID_DIM))
    s2 = prop(hs2, src_g, dst_g)
    out = _m3_call(s2, hs2, dinv, b2.reshape(1, HID_DIM))
    return out
